# baseline (speedup 1.0000x reference)
import jax
import jax.numpy as jnp
from jax.experimental import pallas as pl

NUM_MEANING_TYPES = 26
MEANINGS_PER_TYPE = 64
_FLAT = NUM_MEANING_TYPES * MEANINGS_PER_TYPE


def _onehot_body(o_ref):
    rows, cols = o_ref.shape
    col = jax.lax.broadcasted_iota(jnp.int32, (rows, cols), 1)
    o_ref[...] = jnp.where(col % MEANINGS_PER_TYPE == 0,
                           jnp.float32(1.0), jnp.float32(0.0))


def kernel(utts):
    _, batch = utts.shape
    blk = 4096
    out = pl.pallas_call(
        _onehot_body,
        out_shape=jax.ShapeDtypeStruct((batch, _FLAT), jnp.float32),
        grid=(batch // blk,),
        out_specs=pl.BlockSpec((blk, _FLAT), lambda i: (i, i * 0)),
    )()
    return out.reshape(batch, NUM_MEANING_TYPES, MEANINGS_PER_TYPE)


# speedup vs baseline: 1.0028x; 1.0028x over previous
import jax
import jax.numpy as jnp
from jax.experimental import pallas as pl
from jax.experimental.pallas import tpu as pltpu

NUM_MEANING_TYPES = 26
MEANINGS_PER_TYPE = 64
_FLAT = NUM_MEANING_TYPES * MEANINGS_PER_TYPE


def _onehot_body(o_ref):
    rows, cols = o_ref.shape
    col = jax.lax.broadcasted_iota(jnp.int32, (rows, cols), 1)
    o_ref[...] = jnp.where(col % MEANINGS_PER_TYPE == 0,
                           jnp.float32(1.0), jnp.float32(0.0))


def kernel(utts):
    _, batch = utts.shape
    blk = 4096
    out = pl.pallas_call(
        _onehot_body,
        out_shape=jax.ShapeDtypeStruct((batch, _FLAT), jnp.float32),
        grid=(batch // blk,),
        out_specs=pl.BlockSpec((blk, _FLAT), lambda i: (i, i * 0)),
        compiler_params=pltpu.CompilerParams(
            dimension_semantics=("parallel",)),
    )()
    return out.reshape(batch, NUM_MEANING_TYPES, MEANINGS_PER_TYPE)


# speedup vs baseline: 1.0066x; 1.0038x over previous
import jax
import jax.numpy as jnp
from jax.experimental import pallas as pl
from jax.experimental.pallas import tpu as pltpu

NUM_MEANING_TYPES = 26
MEANINGS_PER_TYPE = 64
_FLAT = NUM_MEANING_TYPES * MEANINGS_PER_TYPE
_TILE_ROWS = 2048


def _onehot_body(o_ref, tile, sems):
    rows, cols = tile.shape
    col = jax.lax.broadcasted_iota(jnp.int32, (rows, cols), 1)
    tile[...] = jnp.where(col % MEANINGS_PER_TYPE == 0,
                          jnp.float32(1.0), jnp.float32(0.0))
    n = o_ref.shape[0] // rows
    for i in range(n):
        pltpu.make_async_copy(
            tile, o_ref.at[pl.ds(jnp.int32(i * rows), rows), :],
            sems.at[jnp.int32(i)]).start()
    for i in range(n):
        pltpu.make_async_copy(
            tile, o_ref.at[pl.ds(jnp.int32(i * rows), rows), :],
            sems.at[jnp.int32(i)]).wait()


def kernel(utts):
    _, batch = utts.shape
    n_copies = batch // _TILE_ROWS
    out = pl.pallas_call(
        _onehot_body,
        out_shape=jax.ShapeDtypeStruct((batch, _FLAT), jnp.float32),
        out_specs=pl.BlockSpec(memory_space=pl.ANY),
        scratch_shapes=[
            pltpu.VMEM((_TILE_ROWS, _FLAT), jnp.float32),
            pltpu.SemaphoreType.DMA((n_copies,)),
        ],
    )()
    return out.reshape(batch, NUM_MEANING_TYPES, MEANINGS_PER_TYPE)
